# Initial kernel scaffold; baseline (speedup 1.0000x reference)
import jax
import jax.numpy as jnp
from jax.experimental import pallas as pl


def kernel(x, edge_index, edge_type, params):
    raise NotImplementedError("write your pallas kernel here")


# speedup vs baseline: 7.2264x; 7.2264x over previous
import functools
import numpy as np
import jax
import jax.numpy as jnp
from jax import lax
from jax.experimental import pallas as pl
from jax.experimental.pallas import tpu as pltpu
from jax.experimental.pallas import tpu_sc as plsc

N = 10000
E = 320000
DIN = 128
H = 256
R = 3

NPAD = 10240
L = 16
NC = 2
NS = 16
TPAD = R * NPAD
COLS = 32
CHUNK = 128
NCH = 158
EP = NCH * CHUNK
EPAD = EP * NS
ZPT = TPAD // NS
RS = float(1.0 / np.sqrt(1.0 + 1e-5))

_BN = 1024
NB = NPAD // _BN

_SC_PARAMS = dict(needs_layout_passes=False, use_tc_tiling_on_sc=False)


def _mesh():
    return plsc.VectorSubcoreMesh(core_axis_name="c", subcore_axis_name="s")


def _sc_pass_body(nq, *refs):
    (tab, gidx, tix3, out,
     acc, gbuf, gix, tix, zbuf, gsem0, gsem1, isem0, isem1) = refs

    c = lax.axis_index("c")
    s = lax.axis_index("s")
    zero16 = jnp.zeros((L,), jnp.float32)

    def zero_zbuf(k, _):
        for cc in range(COLS // L):
            zbuf[k, pl.ds(cc * L, L)] = zero16
        return 0
    lax.fori_loop(0, zbuf.shape[0], zero_zbuf, 0)

    gsems = (gsem0, gsem1)
    isems = (isem0, isem1)

    for q in range(nq):
        kk = c * nq + q
        goff = kk * EPAD + s * EP

        def zero_acc(m, _):
            pltpu.sync_copy(
                zbuf, acc.at[pl.ds(s * ZPT + m * zbuf.shape[0], zbuf.shape[0])])
            return 0
        lax.fori_loop(0, ZPT // zbuf.shape[0], zero_acc, 0)

        plsc.subcore_barrier()

        def ix_descs(j, b):
            return (
                pltpu.make_async_copy(
                    gidx.at[pl.ds(goff + j * CHUNK, CHUNK)], gix.at[b], isems[b]),
                pltpu.make_async_copy(
                    tix3.at[s, j], tix.at[b], isems[b]),
            )

        def g_desc(j, b):
            return pltpu.make_async_copy(
                tab.at[gix.at[b]], gbuf.at[b], gsems[b])

        for d in ix_descs(0, 0):
            d.start()
        for d in ix_descs(0, 0):
            d.wait()
        g_desc(0, 0).start()
        for d in ix_descs(1, 1):
            d.start()

        def chunk_body(jj, _):
            for b in range(2):
                j = 2 * jj + b
                g_desc(j, b).wait()

                @pl.when(j + 1 < NCH)
                def _():
                    for d in ix_descs(j + 1, 1 - b):
                        d.wait()
                    g_desc(j + 1, 1 - b).start()

                pltpu.sync_copy(gbuf.at[b], acc.at[tix.at[b]], add=True)

                @pl.when(j + 2 < NCH)
                def _():
                    for d in ix_descs(j + 2, b):
                        d.start()
            return 0
        lax.fori_loop(0, NCH // 2, chunk_body, 0)

        plsc.subcore_barrier()

        pltpu.sync_copy(
            acc.at[pl.ds(s * ZPT, ZPT)],
            out.at[pl.ds((kk * TPAD + s * ZPT), ZPT)])


def _make_sc_pass(nq):
    return pl.kernel(
        functools.partial(_sc_pass_body, nq),
        out_type=jax.ShapeDtypeStruct((NC * nq * TPAD, COLS), jnp.float32),
        mesh=_mesh(),
        scratch_types=[
            pltpu.VMEM_SHARED((TPAD, COLS), jnp.float32),
            pltpu.VMEM((2, CHUNK, COLS), jnp.float32),
            pltpu.VMEM((2, CHUNK), jnp.int32),
            pltpu.VMEM((2, CHUNK), jnp.int32),
            pltpu.VMEM((64, COLS), jnp.float32),
            pltpu.SemaphoreType.DMA, pltpu.SemaphoreType.DMA,
            pltpu.SemaphoreType.DMA, pltpu.SemaphoreType.DMA,
        ],
        compiler_params=pltpu.CompilerParams(**_SC_PARAMS),
    )


def _sc_cnt_body(tA3, tB3, cnt_out, idxb, cnt_loc):
    c = lax.axis_index("c")
    s = lax.axis_index("s")
    zero16 = jnp.zeros((L,), jnp.float32)
    ones16 = jnp.ones((L,), jnp.float32)

    def zero_cnt(k, _):
        cnt_loc[pl.ds(k * L, L)] = zero16
        return 0
    lax.fori_loop(0, TPAD // L, zero_cnt, 0)

    @pl.when(c == 0)
    def _():
        pltpu.sync_copy(tA3.at[s], idxb)

    @pl.when(c == 1)
    def _():
        pltpu.sync_copy(tB3.at[s], idxb)

    def hist_chunk(j, _):
        for k in range(CHUNK // L):
            idx = idxb[j, pl.ds(k * L, L)]
            plsc.addupdate_scatter(cnt_loc, [idx], ones16)
        return 0
    lax.fori_loop(0, NCH, hist_chunk, 0)

    wid = c * NS + s
    pltpu.sync_copy(cnt_loc, cnt_out.at[pl.ds(wid * TPAD, TPAD)])


def _make_sc_cnt():
    return pl.kernel(
        _sc_cnt_body,
        out_type=jax.ShapeDtypeStruct((NC * NS * TPAD,), jnp.float32),
        mesh=_mesh(),
        scratch_types=[
            pltpu.VMEM((NCH, CHUNK), jnp.int32),
            pltpu.VMEM((TPAD,), jnp.float32),
        ],
        compiler_params=pltpu.CompilerParams(**_SC_PARAMS),
    )


def _full(shape):
    return pl.BlockSpec(shape, lambda i: tuple(0 for _ in shape))


def _sum_specs(nchunks):
    return [
        pl.BlockSpec((nchunks, _BN, COLS),
                     functools.partial(lambda r, i: (0, r * NB + i, 0), r))
        for r in range(R)
    ]


def _cnt_specs():
    return [
        pl.BlockSpec((_BN,),
                     functools.partial(lambda r, i: (r * NB + i,), r))
        for r in range(R)
    ]


def _tc_mlp_body(x, w1, b1, g1, be1, w2, b2, g2, be2, zm, xcat):
    xb = x[...]
    h = jnp.dot(xb, w1[...], preferred_element_type=jnp.float32)
    s1 = g1[...] * RS
    h = jnp.maximum(h * s1 + (b1[...] * s1 + be1[...]), 0.0)
    h = jnp.dot(h, w2[...], preferred_element_type=jnp.float32)
    s2 = g2[...] * RS
    zm[...] = jnp.maximum(h * s2 + (b2[...] * s2 + be2[...]), 0.0)
    for k in range(4):
        xcat[k] = xb[:, k * COLS:(k + 1) * COLS]


def _tc_cnt_body(cin, ca, cb):
    a = cin[0, :]
    b = cin[NS, :]
    for w in range(1, NS):
        a = a + cin[w, :]
        b = b + cin[NS + w, :]
    ca[...] = a
    cb[...] = b


def _tc_hsc_body(x, s0, s1, s2, c0, c1, c2, wroot, b, hscat, r1root):
    xb = x[...]
    ss = (s0, s1, s2)
    cs = (c0, c1, c2)
    ctx = jnp.zeros((_BN, DIN), jnp.float32)
    relcnt = jnp.zeros((_BN,), jnp.float32)
    for r in range(R):
        cr = cs[r][...]
        inv = 1.0 / jnp.maximum(cr, 1.0)
        sr = jnp.concatenate([ss[r][k] for k in range(4)], axis=1)
        ctx = ctx + sr * inv[:, None]
        relcnt = relcnt + jnp.minimum(cr, 1.0)
    h = xb - ctx * (1.0 / jnp.maximum(relcnt, 1.0))[:, None]
    for k in range(4):
        hscat[k] = h[:, k * COLS:(k + 1) * COLS]
    r1root[...] = jnp.dot(h, wroot[...], preferred_element_type=jnp.float32) + b[...]


def _tc_z1_body(r1root, s0, s1, s2, c0, c1, c2,
                wr, g, be, wroot2, b2, z1cat, r2root):
    acc = r1root[...]
    ss = (s0, s1, s2)
    cs = (c0, c1, c2)
    for r in range(R):
        inv = 1.0 / jnp.maximum(cs[r][...], 1.0)
        m = jnp.concatenate([ss[r][k] for k in range(4)], axis=1)
        acc = acc + jnp.dot(m * inv[:, None], wr[r],
                            preferred_element_type=jnp.float32)
    s = g[...] * RS
    z1 = jnp.maximum(acc * s + be[...], 0.0)
    for k in range(8):
        z1cat[k] = z1[:, k * COLS:(k + 1) * COLS]
    r2root[...] = jnp.dot(z1, wroot2[...], preferred_element_type=jnp.float32) + b2[...]


def _tc_out_body(r2root, s0, s1, s2, c0, c1, c2,
                 wr, g, be, zm, ow, ob, logits):
    acc = r2root[...]
    ss = (s0, s1, s2)
    cs = (c0, c1, c2)
    for r in range(R):
        inv = 1.0 / jnp.maximum(cs[r][...], 1.0)
        m = jnp.concatenate([ss[r][k] for k in range(8)], axis=1)
        acc = acc + jnp.dot(m * inv[:, None], wr[r],
                            preferred_element_type=jnp.float32)
    s = g[...] * RS
    z2 = jnp.maximum(acc * s + be[...], 0.0)
    w_all = ow[...]
    out = jnp.dot(zm[...], w_all[:H], preferred_element_type=jnp.float32)
    out = out + jnp.dot(z2, w_all[H:], preferred_element_type=jnp.float32)
    logits[...] = out + ob[...]


def kernel(x, edge_index, edge_type, params):
    p = params
    row = edge_index[0]
    col = edge_index[1]
    et = edge_type

    pad_t = jnp.full((EPAD - E,), N, jnp.int32)
    pad_z = jnp.zeros((EPAD - E,), jnp.int32)
    tA = jnp.concatenate([et * NPAD + row, pad_t]).reshape(NS, NCH, CHUNK)
    tB = jnp.concatenate([et * NPAD + col, pad_t]).reshape(NS, NCH, CHUNK)
    col_p = jnp.concatenate([col, pad_z])
    row_p = jnp.concatenate([row, pad_z])
    gidxA = jnp.concatenate([col_p + k * NPAD for k in range(4)])
    gidxB1 = jnp.concatenate([row_p + k * NPAD for k in range(4)])
    gidxB2 = jnp.concatenate([row_p + k * NPAD for k in range(8)])

    (b1, g1, be1, b2, g2, be2, rb1, rb2) = [
        a.reshape(1, H) for a in (
            p['mlp_b1'], p['mlp_g1'], p['mlp_be1'],
            p['mlp_b2'], p['mlp_g2'], p['mlp_be2'],
            p['rgcn1_b'], p['rgcn2_b'])]
    (bn1g, bn1b, bn2g, bn2b) = [
        a.reshape(1, H) for a in (p['bn1_g'], p['bn1_be'], p['bn2_g'], p['bn2_be'])]

    cnt_raw = _make_sc_cnt()(tA, tB)

    cntA, cntB = pl.pallas_call(
        _tc_cnt_body,
        grid=(TPAD // 3072,),
        in_specs=[pl.BlockSpec((2 * NS, 3072), lambda i: (0, i))],
        out_specs=[pl.BlockSpec((3072,), lambda i: (i,)),
                   pl.BlockSpec((3072,), lambda i: (i,))],
        out_shape=[jax.ShapeDtypeStruct((TPAD,), jnp.float32),
                   jax.ShapeDtypeStruct((TPAD,), jnp.float32)],
    )(cnt_raw.reshape(2 * NS, TPAD))

    zm, xcat = pl.pallas_call(
        _tc_mlp_body,
        grid=(NB,),
        in_specs=[pl.BlockSpec((_BN, DIN), lambda i: (i, 0)),
                  _full((DIN, H))] + [_full((1, H))] * 3 +
                 [_full((H, H))] + [_full((1, H))] * 3,
        out_specs=[pl.BlockSpec((_BN, H), lambda i: (i, 0)),
                   pl.BlockSpec((4, _BN, COLS), lambda i: (0, i, 0))],
        out_shape=[jax.ShapeDtypeStruct((NPAD, H), jnp.float32),
                   jax.ShapeDtypeStruct((4, NPAD, COLS), jnp.float32)],
    )(x, p['mlp_W1'], b1, g1, be1, p['mlp_W2'], b2, g2, be2)

    outA = _make_sc_pass(2)(xcat.reshape(4 * NPAD, COLS), gidxA, tA)
    outA = outA.reshape(4, TPAD, COLS)

    hscat, r1root = pl.pallas_call(
        _tc_hsc_body,
        grid=(NB,),
        in_specs=[pl.BlockSpec((_BN, DIN), lambda i: (i, 0))] +
                 _sum_specs(4) + _cnt_specs() +
                 [_full((DIN, H)), _full((1, H))],
        out_specs=[pl.BlockSpec((4, _BN, COLS), lambda i: (0, i, 0)),
                   pl.BlockSpec((_BN, H), lambda i: (i, 0))],
        out_shape=[jax.ShapeDtypeStruct((4, NPAD, COLS), jnp.float32),
                   jax.ShapeDtypeStruct((NPAD, H), jnp.float32)],
    )(x, outA, outA, outA, cntA, cntA, cntA, p['rgcn1_Wroot'], rb1)

    outB1 = _make_sc_pass(2)(hscat.reshape(4 * NPAD, COLS), gidxB1, tB)
    outB1 = outB1.reshape(4, TPAD, COLS)

    z1cat, r2root = pl.pallas_call(
        _tc_z1_body,
        grid=(NB,),
        in_specs=[pl.BlockSpec((_BN, H), lambda i: (i, 0))] +
                 _sum_specs(4) + _cnt_specs() +
                 [_full((R, DIN, H)), _full((1, H)), _full((1, H)),
                  _full((H, H)), _full((1, H))],
        out_specs=[pl.BlockSpec((8, _BN, COLS), lambda i: (0, i, 0)),
                   pl.BlockSpec((_BN, H), lambda i: (i, 0))],
        out_shape=[jax.ShapeDtypeStruct((8, NPAD, COLS), jnp.float32),
                   jax.ShapeDtypeStruct((NPAD, H), jnp.float32)],
    )(r1root, outB1, outB1, outB1, cntB, cntB, cntB,
      p['rgcn1_Wr'], bn1g, bn1b, p['rgcn2_Wroot'], rb2)

    outB2 = _make_sc_pass(4)(z1cat.reshape(8 * NPAD, COLS), gidxB2, tB)
    outB2 = outB2.reshape(8, TPAD, COLS)

    logits = pl.pallas_call(
        _tc_out_body,
        grid=(NB,),
        in_specs=[pl.BlockSpec((_BN, H), lambda i: (i, 0))] +
                 _sum_specs(8) + _cnt_specs() +
                 [_full((R, H, H)), _full((1, H)), _full((1, H)),
                  pl.BlockSpec((_BN, H), lambda i: (i, 0)),
                  _full((2 * H, 1)), _full((1, 1))],
        out_specs=pl.BlockSpec((_BN, 1), lambda i: (i, 0)),
        out_shape=jax.ShapeDtypeStruct((N, 1), jnp.float32),
    )(r2root, outB2, outB2, outB2, cntB, cntB, cntB,
      p['rgcn2_Wr'], bn2g, bn2b, zm,
      p['out_W'], p['out_b'].reshape(1, 1))

    return logits[:, 0]


# speedup vs baseline: 7.6950x; 1.0649x over previous
import functools
import numpy as np
import jax
import jax.numpy as jnp
from jax import lax
from jax.experimental import pallas as pl
from jax.experimental.pallas import tpu as pltpu
from jax.experimental.pallas import tpu_sc as plsc

N = 10000
E = 320000
DIN = 128
H = 256
R = 3

NPAD = 10240
L = 16
NC = 2
NS = 16
TPAD = R * NPAD
COLS = 32
CHUNK = 128
NBUF = 8
NIB = 16
NCH = 160
EP = NCH * CHUNK
EPAD = EP * NS
ZPT = TPAD // NS
RS = float(1.0 / np.sqrt(1.0 + 1e-5))

_BN = 1024
NB = NPAD // _BN

_SC_PARAMS = dict(needs_layout_passes=False, use_tc_tiling_on_sc=False)


def _mesh():
    return plsc.VectorSubcoreMesh(core_axis_name="c", subcore_axis_name="s")


def _sc_pass_body(nq, *refs):
    (tab, gidx, tix3, out,
     acc, gbuf, gix, tix, zbuf, gsems, isems) = refs

    c = lax.axis_index("c")
    s = lax.axis_index("s")
    zero16 = jnp.zeros((L,), jnp.float32)

    def zero_zbuf(k, _):
        for cc in range(COLS // L):
            zbuf[k, pl.ds(cc * L, L)] = zero16
        return 0
    lax.fori_loop(0, zbuf.shape[0], zero_zbuf, 0)

    for q in range(nq):
        kk = c * nq + q
        goff = kk * EPAD + s * EP

        def zero_acc(m, _):
            pltpu.sync_copy(
                zbuf, acc.at[pl.ds(s * ZPT + m * zbuf.shape[0], zbuf.shape[0])])
            return 0
        lax.fori_loop(0, ZPT // zbuf.shape[0], zero_acc, 0)

        plsc.subcore_barrier()

        def ix_descs(j, sl):
            return (
                pltpu.make_async_copy(
                    gidx.at[pl.ds(goff + j * CHUNK, CHUNK)], gix.at[sl],
                    isems.at[sl]),
                pltpu.make_async_copy(
                    tix3.at[s, j], tix.at[sl], isems.at[sl]),
            )

        def g_desc(sl, b):
            return pltpu.make_async_copy(
                tab.at[gix.at[sl]], gbuf.at[b], gsems.at[b])

        for m in range(NIB - 1):
            for d in ix_descs(m, m):
                d.start()
        for m in range(NBUF - 1):
            for d in ix_descs(m, m):
                d.wait()
            g_desc(m, m).start()

        def chunk_body(jj, _):
            for u in range(NIB):
                j = jj * NIB + u
                b = u % NBUF
                g_desc(u, b).wait()
                pltpu.sync_copy(gbuf.at[b], acc.at[tix.at[u]], add=True)

                sg = (u - 1) % NBUF
                si = (u + NBUF - 1) % NIB

                @pl.when(j + NBUF - 1 < NCH)
                def _():
                    for d in ix_descs(j + NBUF - 1, si):
                        d.wait()
                    g_desc(si, sg).start()

                sd = (u - 1) % NIB

                @pl.when(j + NIB - 1 < NCH)
                def _():
                    for d in ix_descs(j + NIB - 1, sd):
                        d.start()
            return 0
        lax.fori_loop(0, NCH // NIB, chunk_body, 0)

        plsc.subcore_barrier()

        pltpu.sync_copy(
            acc.at[pl.ds(s * ZPT, ZPT)],
            out.at[pl.ds((kk * TPAD + s * ZPT), ZPT)])


def _make_sc_pass(nq):
    return pl.kernel(
        functools.partial(_sc_pass_body, nq),
        out_type=jax.ShapeDtypeStruct((NC * nq * TPAD, COLS), jnp.float32),
        mesh=_mesh(),
        scratch_types=[
            pltpu.VMEM_SHARED((TPAD, COLS), jnp.float32),
            pltpu.VMEM((NBUF, CHUNK, COLS), jnp.float32),
            pltpu.VMEM((NIB, CHUNK), jnp.int32),
            pltpu.VMEM((NIB, CHUNK), jnp.int32),
            pltpu.VMEM((64, COLS), jnp.float32),
            pltpu.SemaphoreType.DMA((NBUF,)),
            pltpu.SemaphoreType.DMA((NIB,)),
        ],
        compiler_params=pltpu.CompilerParams(**_SC_PARAMS),
    )


def _sc_cnt_body(tA3, tB3, cnt_out, idxb, cnt_loc):
    c = lax.axis_index("c")
    s = lax.axis_index("s")
    zero16 = jnp.zeros((L,), jnp.float32)
    ones16 = jnp.ones((L,), jnp.float32)

    def zero_cnt(k, _):
        cnt_loc[pl.ds(k * L, L)] = zero16
        return 0
    lax.fori_loop(0, TPAD // L, zero_cnt, 0)

    @pl.when(c == 0)
    def _():
        pltpu.sync_copy(tA3.at[s], idxb)

    @pl.when(c == 1)
    def _():
        pltpu.sync_copy(tB3.at[s], idxb)

    def hist_chunk(j, _):
        for k in range(CHUNK // L):
            idx = idxb[j, pl.ds(k * L, L)]
            plsc.addupdate_scatter(cnt_loc, [idx], ones16)
        return 0
    lax.fori_loop(0, NCH, hist_chunk, 0)

    wid = c * NS + s
    pltpu.sync_copy(cnt_loc, cnt_out.at[pl.ds(wid * TPAD, TPAD)])


def _make_sc_cnt():
    return pl.kernel(
        _sc_cnt_body,
        out_type=jax.ShapeDtypeStruct((NC * NS * TPAD,), jnp.float32),
        mesh=_mesh(),
        scratch_types=[
            pltpu.VMEM((NCH, CHUNK), jnp.int32),
            pltpu.VMEM((TPAD,), jnp.float32),
        ],
        compiler_params=pltpu.CompilerParams(**_SC_PARAMS),
    )


def _full(shape):
    return pl.BlockSpec(shape, lambda i: tuple(0 for _ in shape))


def _sum_specs(nchunks):
    return [
        pl.BlockSpec((nchunks, _BN, COLS),
                     functools.partial(lambda r, i: (0, r * NB + i, 0), r))
        for r in range(R)
    ]


def _cnt_specs():
    return [
        pl.BlockSpec((_BN,),
                     functools.partial(lambda r, i: (r * NB + i,), r))
        for r in range(R)
    ]


def _tc_mlp_body(x, w1, b1, g1, be1, w2, b2, g2, be2, zm, xcat):
    xb = x[...]
    h = jnp.dot(xb, w1[...], preferred_element_type=jnp.float32)
    s1 = g1[...] * RS
    h = jnp.maximum(h * s1 + (b1[...] * s1 + be1[...]), 0.0)
    h = jnp.dot(h, w2[...], preferred_element_type=jnp.float32)
    s2 = g2[...] * RS
    zm[...] = jnp.maximum(h * s2 + (b2[...] * s2 + be2[...]), 0.0)
    for k in range(4):
        xcat[k] = xb[:, k * COLS:(k + 1) * COLS]


def _tc_cnt_body(cin, ca, cb):
    a = cin[0, :]
    b = cin[NS, :]
    for w in range(1, NS):
        a = a + cin[w, :]
        b = b + cin[NS + w, :]
    ca[...] = a
    cb[...] = b


def _tc_hsc_body(x, s0, s1, s2, c0, c1, c2, wroot, b, hscat, r1root):
    xb = x[...]
    ss = (s0, s1, s2)
    cs = (c0, c1, c2)
    ctx = jnp.zeros((_BN, DIN), jnp.float32)
    relcnt = jnp.zeros((_BN,), jnp.float32)
    for r in range(R):
        cr = cs[r][...]
        inv = 1.0 / jnp.maximum(cr, 1.0)
        sr = jnp.concatenate([ss[r][k] for k in range(4)], axis=1)
        ctx = ctx + sr * inv[:, None]
        relcnt = relcnt + jnp.minimum(cr, 1.0)
    h = xb - ctx * (1.0 / jnp.maximum(relcnt, 1.0))[:, None]
    for k in range(4):
        hscat[k] = h[:, k * COLS:(k + 1) * COLS]
    r1root[...] = jnp.dot(h, wroot[...], preferred_element_type=jnp.float32) + b[...]


def _tc_z1_body(r1root, s0, s1, s2, c0, c1, c2,
                wr, g, be, wroot2, b2, z1cat, r2root):
    acc = r1root[...]
    ss = (s0, s1, s2)
    cs = (c0, c1, c2)
    for r in range(R):
        inv = 1.0 / jnp.maximum(cs[r][...], 1.0)
        m = jnp.concatenate([ss[r][k] for k in range(4)], axis=1)
        acc = acc + jnp.dot(m * inv[:, None], wr[r],
                            preferred_element_type=jnp.float32)
    s = g[...] * RS
    z1 = jnp.maximum(acc * s + be[...], 0.0)
    for k in range(8):
        z1cat[k] = z1[:, k * COLS:(k + 1) * COLS]
    r2root[...] = jnp.dot(z1, wroot2[...], preferred_element_type=jnp.float32) + b2[...]


def _tc_out_body(r2root, s0, s1, s2, c0, c1, c2,
                 wr, g, be, zm, ow, ob, logits):
    acc = r2root[...]
    ss = (s0, s1, s2)
    cs = (c0, c1, c2)
    for r in range(R):
        inv = 1.0 / jnp.maximum(cs[r][...], 1.0)
        m = jnp.concatenate([ss[r][k] for k in range(8)], axis=1)
        acc = acc + jnp.dot(m * inv[:, None], wr[r],
                            preferred_element_type=jnp.float32)
    s = g[...] * RS
    z2 = jnp.maximum(acc * s + be[...], 0.0)
    w_all = ow[...]
    out = jnp.dot(zm[...], w_all[:H], preferred_element_type=jnp.float32)
    out = out + jnp.dot(z2, w_all[H:], preferred_element_type=jnp.float32)
    logits[...] = out + ob[...]


def kernel(x, edge_index, edge_type, params):
    p = params
    row = edge_index[0]
    col = edge_index[1]
    et = edge_type

    pad_t = jnp.full((EPAD - E,), N, jnp.int32)
    pad_z = jnp.zeros((EPAD - E,), jnp.int32)
    tA = jnp.concatenate([et * NPAD + row, pad_t]).reshape(NS, NCH, CHUNK)
    tB = jnp.concatenate([et * NPAD + col, pad_t]).reshape(NS, NCH, CHUNK)
    col_p = jnp.concatenate([col, pad_z])
    row_p = jnp.concatenate([row, pad_z])
    gidxA = jnp.concatenate([col_p + k * NPAD for k in range(4)])
    gidxB1 = jnp.concatenate([row_p + k * NPAD for k in range(4)])
    gidxB2 = jnp.concatenate([row_p + k * NPAD for k in range(8)])

    (b1, g1, be1, b2, g2, be2, rb1, rb2) = [
        a.reshape(1, H) for a in (
            p['mlp_b1'], p['mlp_g1'], p['mlp_be1'],
            p['mlp_b2'], p['mlp_g2'], p['mlp_be2'],
            p['rgcn1_b'], p['rgcn2_b'])]
    (bn1g, bn1b, bn2g, bn2b) = [
        a.reshape(1, H) for a in (p['bn1_g'], p['bn1_be'], p['bn2_g'], p['bn2_be'])]

    cnt_raw = _make_sc_cnt()(tA, tB)

    cntA, cntB = pl.pallas_call(
        _tc_cnt_body,
        grid=(TPAD // 3072,),
        in_specs=[pl.BlockSpec((2 * NS, 3072), lambda i: (0, i))],
        out_specs=[pl.BlockSpec((3072,), lambda i: (i,)),
                   pl.BlockSpec((3072,), lambda i: (i,))],
        out_shape=[jax.ShapeDtypeStruct((TPAD,), jnp.float32),
                   jax.ShapeDtypeStruct((TPAD,), jnp.float32)],
    )(cnt_raw.reshape(2 * NS, TPAD))

    zm, xcat = pl.pallas_call(
        _tc_mlp_body,
        grid=(NB,),
        in_specs=[pl.BlockSpec((_BN, DIN), lambda i: (i, 0)),
                  _full((DIN, H))] + [_full((1, H))] * 3 +
                 [_full((H, H))] + [_full((1, H))] * 3,
        out_specs=[pl.BlockSpec((_BN, H), lambda i: (i, 0)),
                   pl.BlockSpec((4, _BN, COLS), lambda i: (0, i, 0))],
        out_shape=[jax.ShapeDtypeStruct((NPAD, H), jnp.float32),
                   jax.ShapeDtypeStruct((4, NPAD, COLS), jnp.float32)],
    )(x, p['mlp_W1'], b1, g1, be1, p['mlp_W2'], b2, g2, be2)

    outA = _make_sc_pass(2)(xcat.reshape(4 * NPAD, COLS), gidxA, tA)
    outA = outA.reshape(4, TPAD, COLS)

    hscat, r1root = pl.pallas_call(
        _tc_hsc_body,
        grid=(NB,),
        in_specs=[pl.BlockSpec((_BN, DIN), lambda i: (i, 0))] +
                 _sum_specs(4) + _cnt_specs() +
                 [_full((DIN, H)), _full((1, H))],
        out_specs=[pl.BlockSpec((4, _BN, COLS), lambda i: (0, i, 0)),
                   pl.BlockSpec((_BN, H), lambda i: (i, 0))],
        out_shape=[jax.ShapeDtypeStruct((4, NPAD, COLS), jnp.float32),
                   jax.ShapeDtypeStruct((NPAD, H), jnp.float32)],
    )(x, outA, outA, outA, cntA, cntA, cntA, p['rgcn1_Wroot'], rb1)

    outB1 = _make_sc_pass(2)(hscat.reshape(4 * NPAD, COLS), gidxB1, tB)
    outB1 = outB1.reshape(4, TPAD, COLS)

    z1cat, r2root = pl.pallas_call(
        _tc_z1_body,
        grid=(NB,),
        in_specs=[pl.BlockSpec((_BN, H), lambda i: (i, 0))] +
                 _sum_specs(4) + _cnt_specs() +
                 [_full((R, DIN, H)), _full((1, H)), _full((1, H)),
                  _full((H, H)), _full((1, H))],
        out_specs=[pl.BlockSpec((8, _BN, COLS), lambda i: (0, i, 0)),
                   pl.BlockSpec((_BN, H), lambda i: (i, 0))],
        out_shape=[jax.ShapeDtypeStruct((8, NPAD, COLS), jnp.float32),
                   jax.ShapeDtypeStruct((NPAD, H), jnp.float32)],
    )(r1root, outB1, outB1, outB1, cntB, cntB, cntB,
      p['rgcn1_Wr'], bn1g, bn1b, p['rgcn2_Wroot'], rb2)

    outB2 = _make_sc_pass(4)(z1cat.reshape(8 * NPAD, COLS), gidxB2, tB)
    outB2 = outB2.reshape(8, TPAD, COLS)

    logits = pl.pallas_call(
        _tc_out_body,
        grid=(NB,),
        in_specs=[pl.BlockSpec((_BN, H), lambda i: (i, 0))] +
                 _sum_specs(8) + _cnt_specs() +
                 [_full((R, H, H)), _full((1, H)), _full((1, H)),
                  pl.BlockSpec((_BN, H), lambda i: (i, 0)),
                  _full((2 * H, 1)), _full((1, 1))],
        out_specs=pl.BlockSpec((_BN, 1), lambda i: (i, 0)),
        out_shape=jax.ShapeDtypeStruct((N, 1), jnp.float32),
    )(r2root, outB2, outB2, outB2, cntB, cntB, cntB,
      p['rgcn2_Wr'], bn2g, bn2b, zm,
      p['out_W'], p['out_b'].reshape(1, 1))

    return logits[:, 0]


# speedup vs baseline: 7.9602x; 1.0345x over previous
import functools
import numpy as np
import jax
import jax.numpy as jnp
from jax import lax
from jax.experimental import pallas as pl
from jax.experimental.pallas import tpu as pltpu
from jax.experimental.pallas import tpu_sc as plsc

N = 10000
E = 320000
DIN = 128
H = 256
R = 3

NPAD = 10240
L = 16
NC = 2
NS = 16
TPAD = R * NPAD
COLS = 32
CHUNK = 128
NBUF = 8
NIB = 16
NCH = 160
EP = NCH * CHUNK
EPAD = EP * NS
ZPT = TPAD // NS
RS = float(1.0 / np.sqrt(1.0 + 1e-5))

_BN = 1024
NB = NPAD // _BN

_SC_PARAMS = dict(needs_layout_passes=False, use_tc_tiling_on_sc=False)


def _mesh():
    return plsc.VectorSubcoreMesh(core_axis_name="c", subcore_axis_name="s")


def _sc_pass_body(nq, do_hist, *refs):
    if do_hist:
        (tab, gidx, tix3, hb3, out, cnt_out,
         acc, gbuf, gix, tix, zbuf, hbuf, cnt_loc, gsems, isems, ssems) = refs
    else:
        (tab, gidx, tix3, out,
         acc, gbuf, gix, tix, zbuf, gsems, isems, ssems) = refs

    c = lax.axis_index("c")
    s = lax.axis_index("s")
    zero16 = jnp.zeros((L,), jnp.float32)
    ones16 = jnp.ones((L,), jnp.float32)

    if do_hist:
        def zero_cnt(k, _):
            cnt_loc[pl.ds(k * L, L)] = zero16
            return 0
        lax.fori_loop(0, TPAD // L, zero_cnt, 0)

    def zero_zbuf(k, _):
        for cc in range(COLS // L):
            zbuf[k, pl.ds(cc * L, L)] = zero16
        return 0
    lax.fori_loop(0, zbuf.shape[0], zero_zbuf, 0)

    for q in range(nq):
        kk = c * nq + q
        goff = kk * EPAD + s * EP

        def zero_acc(m, _):
            pltpu.sync_copy(
                zbuf, acc.at[pl.ds(s * ZPT + m * zbuf.shape[0], zbuf.shape[0])])
            return 0
        lax.fori_loop(0, ZPT // zbuf.shape[0], zero_acc, 0)

        plsc.subcore_barrier()

        def ix_descs(j, sl):
            return (
                pltpu.make_async_copy(
                    gidx.at[pl.ds(goff + j * CHUNK, CHUNK)], gix.at[sl],
                    isems.at[sl]),
                pltpu.make_async_copy(
                    tix3.at[s, j], tix.at[sl], isems.at[sl]),
            )

        def g_desc(sl, b):
            return pltpu.make_async_copy(
                tab.at[gix.at[sl]], gbuf.at[b], gsems.at[b])

        def s_start(u, b):
            pltpu.async_copy(
                gbuf.at[b], acc.at[tix.at[u]], ssems.at[b], add=True)

        def s_wait(u, b):
            pltpu.make_async_copy(
                gbuf.at[b], acc.at[tix.at[u]], ssems.at[b]).wait()

        for m in range(NIB - 3):
            for d in ix_descs(m, m):
                d.start()
        for m in range(NBUF - 3):
            for d in ix_descs(m, m):
                d.wait()
            g_desc(m, m).start()

        def chunk_body(jj, _):
            for u in range(NIB):
                j = jj * NIB + u
                b = u % NBUF
                g_desc(u, b).wait()
                pltpu.sync_copy(gbuf.at[b], acc.at[tix.at[u]], add=True)

                if do_hist and q == 0:
                    @pl.when(c == 0)
                    def _():
                        for k in range(CHUNK // L):
                            idx = tix[u, pl.ds(k * L, L)]
                            plsc.addupdate_scatter(cnt_loc, [idx], ones16)

                    @pl.when(c == 1)
                    def _():
                        pltpu.sync_copy(hb3.at[s, j], hbuf)
                        for k in range(CHUNK // L):
                            idx = hbuf[pl.ds(k * L, L)]
                            plsc.addupdate_scatter(cnt_loc, [idx], ones16)

                sg = (u - 3) % NBUF
                si = (u + NBUF - 3) % NIB
                mg = j + NBUF - 3

                @pl.when(mg < NCH)
                def _():
                    for d in ix_descs(mg, si):
                        d.wait()
                    g_desc(si, sg).start()

                sd = (u - 3) % NIB

                @pl.when(j + NIB - 3 < NCH)
                def _():
                    for d in ix_descs(j + NIB - 3, sd):
                        d.start()
            return 0
        lax.fori_loop(0, NCH // NIB, chunk_body, 0)

        plsc.subcore_barrier()

        pltpu.sync_copy(
            acc.at[pl.ds(s * ZPT, ZPT)],
            out.at[pl.ds((kk * TPAD + s * ZPT), ZPT)])

    if do_hist:
        wid = c * NS + s
        pltpu.sync_copy(cnt_loc, cnt_out.at[pl.ds(wid * TPAD, TPAD)])


def _make_sc_pass(nq, do_hist=False):
    out_types = [jax.ShapeDtypeStruct((NC * nq * TPAD, COLS), jnp.float32)]
    scratch = [
        pltpu.VMEM_SHARED((TPAD, COLS), jnp.float32),
        pltpu.VMEM((NBUF, CHUNK, COLS), jnp.float32),
        pltpu.VMEM((NIB, CHUNK), jnp.int32),
        pltpu.VMEM((NIB, CHUNK), jnp.int32),
        pltpu.VMEM((32, COLS), jnp.float32),
    ]
    if do_hist:
        out_types.append(jax.ShapeDtypeStruct((NC * NS * TPAD,), jnp.float32))
        scratch.append(pltpu.VMEM((CHUNK,), jnp.int32))
        scratch.append(pltpu.VMEM((TPAD,), jnp.float32))
    scratch += [
        pltpu.SemaphoreType.DMA((NBUF,)),
        pltpu.SemaphoreType.DMA((NIB,)),
        pltpu.SemaphoreType.DMA((NBUF,)),
    ]
    return pl.kernel(
        functools.partial(_sc_pass_body, nq, do_hist),
        out_type=tuple(out_types) if do_hist else out_types[0],
        mesh=_mesh(),
        scratch_types=scratch,
        compiler_params=pltpu.CompilerParams(**_SC_PARAMS),
    )


def _full(shape):
    return pl.BlockSpec(shape, lambda i: tuple(0 for _ in shape))


def _sum_specs(nchunks):
    return [
        pl.BlockSpec((nchunks, _BN, COLS),
                     functools.partial(lambda r, i: (0, r * NB + i, 0), r))
        for r in range(R)
    ]


def _tc_xcat_body(x, xcat):
    xb = x[...]
    for k in range(4):
        xcat[k] = xb[:, k * COLS:(k + 1) * COLS]


def _tc_hsc_body(x, s0, s1, s2, cr0, cr1, cr2, wroot, b,
                 hscat, r1root, cntB):
    xb = x[...]
    ss = (s0, s1, s2)
    craw = (cr0, cr1, cr2)
    ctx = jnp.zeros((_BN, DIN), jnp.float32)
    relcnt = jnp.zeros((_BN,), jnp.float32)
    cb = []
    for r in range(R):
        blk = craw[r][...]
        ca = blk[0]
        cbr = blk[NS]
        for w in range(1, NS):
            ca = ca + blk[w]
            cbr = cbr + blk[NS + w]
        cb.append(cbr)
        inv = 1.0 / jnp.maximum(ca, 1.0)
        sr = jnp.concatenate([ss[r][k] for k in range(4)], axis=1)
        ctx = ctx + sr * inv[:, None]
        relcnt = relcnt + jnp.minimum(ca, 1.0)
    cntB[...] = jnp.stack(cb, axis=0)
    h = xb - ctx * (1.0 / jnp.maximum(relcnt, 1.0))[:, None]
    for k in range(4):
        hscat[k] = h[:, k * COLS:(k + 1) * COLS]
    r1root[...] = jnp.dot(h, wroot[...], preferred_element_type=jnp.float32) + b[...]


def _tc_z1_body(r1root, s0, s1, s2, cb, wr, g, be, wroot2, b2,
                z1cat, r2root):
    acc = r1root[...]
    ss = (s0, s1, s2)
    cbb = cb[...]
    for r in range(R):
        inv = 1.0 / jnp.maximum(cbb[r], 1.0)
        m = jnp.concatenate([ss[r][k] for k in range(4)], axis=1)
        acc = acc + jnp.dot(m * inv[:, None], wr[r],
                            preferred_element_type=jnp.float32)
    s = g[...] * RS
    z1 = jnp.maximum(acc * s + be[...], 0.0)
    for k in range(8):
        z1cat[k] = z1[:, k * COLS:(k + 1) * COLS]
    r2root[...] = jnp.dot(z1, wroot2[...], preferred_element_type=jnp.float32) + b2[...]


def _tc_out_body(r2root, s0, s1, s2, cb, wr, g, be,
                 x, w1, b1, g1, be1, w2, b2, g2, be2,
                 ow, ob, logits):
    acc = r2root[...]
    ss = (s0, s1, s2)
    cbb = cb[...]
    for r in range(R):
        inv = 1.0 / jnp.maximum(cbb[r], 1.0)
        m = jnp.concatenate([ss[r][k] for k in range(8)], axis=1)
        acc = acc + jnp.dot(m * inv[:, None], wr[r],
                            preferred_element_type=jnp.float32)
    s2c = g[...] * RS
    z2 = jnp.maximum(acc * s2c + be[...], 0.0)

    hm = jnp.dot(x[...], w1[...], preferred_element_type=jnp.float32)
    sm1 = g1[...] * RS
    hm = jnp.maximum(hm * sm1 + (b1[...] * sm1 + be1[...]), 0.0)
    hm = jnp.dot(hm, w2[...], preferred_element_type=jnp.float32)
    sm2 = g2[...] * RS
    zm = jnp.maximum(hm * sm2 + (b2[...] * sm2 + be2[...]), 0.0)

    w_all = ow[...]
    out = jnp.dot(zm, w_all[:H], preferred_element_type=jnp.float32)
    out = out + jnp.dot(z2, w_all[H:], preferred_element_type=jnp.float32)
    logits[...] = out + ob[...]


def kernel(x, edge_index, edge_type, params):
    p = params
    row = edge_index[0]
    col = edge_index[1]
    et = edge_type

    pad_t = jnp.full((EPAD - E,), N, jnp.int32)
    pad_z = jnp.zeros((EPAD - E,), jnp.int32)
    tA = jnp.concatenate([et * NPAD + row, pad_t]).reshape(NS, NCH, CHUNK)
    tB = jnp.concatenate([et * NPAD + col, pad_t]).reshape(NS, NCH, CHUNK)
    col_p = jnp.concatenate([col, pad_z])
    row_p = jnp.concatenate([row, pad_z])
    gidxA = jnp.concatenate([col_p + k * NPAD for k in range(4)])
    gidxB1 = jnp.concatenate([row_p + k * NPAD for k in range(4)])
    gidxB2 = jnp.concatenate([row_p + k * NPAD for k in range(8)])

    (b1, g1, be1, b2, g2, be2, rb1, rb2) = [
        a.reshape(1, H) for a in (
            p['mlp_b1'], p['mlp_g1'], p['mlp_be1'],
            p['mlp_b2'], p['mlp_g2'], p['mlp_be2'],
            p['rgcn1_b'], p['rgcn2_b'])]
    (bn1g, bn1b, bn2g, bn2b) = [
        a.reshape(1, H) for a in (p['bn1_g'], p['bn1_be'], p['bn2_g'], p['bn2_be'])]

    xcat = pl.pallas_call(
        _tc_xcat_body,
        grid=(NB,),
        in_specs=[pl.BlockSpec((_BN, DIN), lambda i: (i, 0))],
        out_specs=pl.BlockSpec((4, _BN, COLS), lambda i: (0, i, 0)),
        out_shape=jax.ShapeDtypeStruct((4, NPAD, COLS), jnp.float32),
    )(x)

    outA, cnt_raw = _make_sc_pass(2, True)(
        xcat.reshape(4 * NPAD, COLS), gidxA, tA, tB)
    outA = outA.reshape(4, TPAD, COLS)
    cnt_raw = cnt_raw.reshape(2 * NS, TPAD)

    craw_specs = [
        pl.BlockSpec((2 * NS, _BN),
                     functools.partial(lambda r, i: (0, r * NB + i), r))
        for r in range(R)
    ]
    hscat, r1root, cntB = pl.pallas_call(
        _tc_hsc_body,
        grid=(NB,),
        in_specs=[pl.BlockSpec((_BN, DIN), lambda i: (i, 0))] +
                 _sum_specs(4) + craw_specs +
                 [_full((DIN, H)), _full((1, H))],
        out_specs=[pl.BlockSpec((4, _BN, COLS), lambda i: (0, i, 0)),
                   pl.BlockSpec((_BN, H), lambda i: (i, 0)),
                   pl.BlockSpec((R, _BN), lambda i: (0, i))],
        out_shape=[jax.ShapeDtypeStruct((4, NPAD, COLS), jnp.float32),
                   jax.ShapeDtypeStruct((NPAD, H), jnp.float32),
                   jax.ShapeDtypeStruct((R, NPAD), jnp.float32)],
    )(x, outA, outA, outA, cnt_raw, cnt_raw, cnt_raw, p['rgcn1_Wroot'], rb1)

    outB1 = _make_sc_pass(2)(hscat.reshape(4 * NPAD, COLS), gidxB1, tB)
    outB1 = outB1.reshape(4, TPAD, COLS)

    cntB_spec = pl.BlockSpec((R, _BN), lambda i: (0, i))

    z1cat, r2root = pl.pallas_call(
        _tc_z1_body,
        grid=(NB,),
        in_specs=[pl.BlockSpec((_BN, H), lambda i: (i, 0))] +
                 _sum_specs(4) + [cntB_spec] +
                 [_full((R, DIN, H)), _full((1, H)), _full((1, H)),
                  _full((H, H)), _full((1, H))],
        out_specs=[pl.BlockSpec((8, _BN, COLS), lambda i: (0, i, 0)),
                   pl.BlockSpec((_BN, H), lambda i: (i, 0))],
        out_shape=[jax.ShapeDtypeStruct((8, NPAD, COLS), jnp.float32),
                   jax.ShapeDtypeStruct((NPAD, H), jnp.float32)],
    )(r1root, outB1, outB1, outB1, cntB,
      p['rgcn1_Wr'], bn1g, bn1b, p['rgcn2_Wroot'], rb2)

    outB2 = _make_sc_pass(4)(z1cat.reshape(8 * NPAD, COLS), gidxB2, tB)
    outB2 = outB2.reshape(8, TPAD, COLS)

    logits = pl.pallas_call(
        _tc_out_body,
        grid=(NB,),
        in_specs=[pl.BlockSpec((_BN, H), lambda i: (i, 0))] +
                 _sum_specs(8) + [cntB_spec] +
                 [_full((R, H, H)), _full((1, H)), _full((1, H)),
                  pl.BlockSpec((_BN, DIN), lambda i: (i, 0)),
                  _full((DIN, H))] + [_full((1, H))] * 3 +
                 [_full((H, H))] + [_full((1, H))] * 3 +
                 [_full((2 * H, 1)), _full((1, 1))],
        out_specs=pl.BlockSpec((_BN, 1), lambda i: (i, 0)),
        out_shape=jax.ShapeDtypeStruct((N, 1), jnp.float32),
    )(r2root, outB2, outB2, outB2, cntB,
      p['rgcn2_Wr'], bn2g, bn2b,
      x, p['mlp_W1'], b1, g1, be1, p['mlp_W2'], b2, g2, be2,
      p['out_W'], p['out_b'].reshape(1, 1))

    return logits[:, 0]


# speedup vs baseline: 13.1936x; 1.6575x over previous
import functools
import numpy as np
import jax
import jax.numpy as jnp
from jax import lax
from jax.experimental import pallas as pl
from jax.experimental.pallas import tpu as pltpu
from jax.experimental.pallas import tpu_sc as plsc

N = 10000
E = 320000
DIN = 128
H = 256
R = 3

NPAD = 10240
L = 16
NC = 2
NS = 16
TPAD = R * NPAD
COLS = 64
SCDT = jnp.bfloat16
NCK1 = DIN // COLS
NCK2 = H // COLS
CHUNK = 128
NBUF = 8
NIB = 16
NCH = 160
EP = NCH * CHUNK
EPAD = EP * NS
ZPT = TPAD // NS
RS = float(1.0 / np.sqrt(1.0 + 1e-5))

_BN = 1024
NB = NPAD // _BN

_SC_PARAMS = dict(needs_layout_passes=False, use_tc_tiling_on_sc=False)


def _mesh():
    return plsc.VectorSubcoreMesh(core_axis_name="c", subcore_axis_name="s")


def _sc_pass_body(nq, do_hist, *refs):
    if do_hist:
        (tab, gidx, tix3, hb3, out, cnt_out,
         acc, gbuf, gix, tix, zbuf, hbuf, cnt_loc, gsems, isems, ssems) = refs
    else:
        (tab, gidx, tix3, out,
         acc, gbuf, gix, tix, zbuf, gsems, isems, ssems) = refs

    c = lax.axis_index("c")
    s = lax.axis_index("s")
    zero16 = jnp.zeros((L,), jnp.float32)
    zero32b = jnp.zeros((2 * L,), SCDT)
    ones16 = jnp.ones((L,), jnp.float32)

    if do_hist:
        def zero_cnt(k, _):
            cnt_loc[pl.ds(k * L, L)] = zero16
            return 0
        lax.fori_loop(0, TPAD // L, zero_cnt, 0)

    def zero_zbuf(k, _):
        for cc in range(COLS // (2 * L)):
            zbuf[k, pl.ds(cc * 2 * L, 2 * L)] = zero32b
        return 0
    lax.fori_loop(0, zbuf.shape[0], zero_zbuf, 0)

    for q in range(nq):
        kk = c * nq + q
        goff = kk * EPAD + s * EP

        def zero_acc(m, _):
            pltpu.sync_copy(
                zbuf, acc.at[pl.ds(s * ZPT + m * zbuf.shape[0], zbuf.shape[0])])
            return 0
        lax.fori_loop(0, ZPT // zbuf.shape[0], zero_acc, 0)

        plsc.subcore_barrier()

        def ix_descs(j, sl):
            return (
                pltpu.make_async_copy(
                    gidx.at[pl.ds(goff + j * CHUNK, CHUNK)], gix.at[sl],
                    isems.at[sl]),
                pltpu.make_async_copy(
                    tix3.at[s, j], tix.at[sl], isems.at[sl]),
            )

        def g_desc(sl, b):
            return pltpu.make_async_copy(
                tab.at[gix.at[sl]], gbuf.at[b], gsems.at[b])

        def s_start(u, b):
            pltpu.async_copy(
                gbuf.at[b], acc.at[tix.at[u]], ssems.at[b], add=True)

        def s_wait(u, b):
            pltpu.make_async_copy(
                gbuf.at[b], acc.at[tix.at[u]], ssems.at[b]).wait()

        for m in range(NIB - 3):
            for d in ix_descs(m, m):
                d.start()
        for m in range(NBUF - 3):
            for d in ix_descs(m, m):
                d.wait()
            g_desc(m, m).start()

        def chunk_body(jj, _):
            for u in range(NIB):
                j = jj * NIB + u
                b = u % NBUF
                g_desc(u, b).wait()
                pltpu.sync_copy(gbuf.at[b], acc.at[tix.at[u]], add=True)

                if do_hist and q == 0:
                    @pl.when(c == 0)
                    def _():
                        for k in range(CHUNK // L):
                            idx = tix[u, pl.ds(k * L, L)]
                            plsc.addupdate_scatter(cnt_loc, [idx], ones16)

                    @pl.when(c == 1)
                    def _():
                        pltpu.sync_copy(hb3.at[s, j], hbuf)
                        for k in range(CHUNK // L):
                            idx = hbuf[pl.ds(k * L, L)]
                            plsc.addupdate_scatter(cnt_loc, [idx], ones16)

                sg = (u - 3) % NBUF
                si = (u + NBUF - 3) % NIB
                mg = j + NBUF - 3

                @pl.when(mg < NCH)
                def _():
                    for d in ix_descs(mg, si):
                        d.wait()
                    g_desc(si, sg).start()

                sd = (u - 3) % NIB

                @pl.when(j + NIB - 3 < NCH)
                def _():
                    for d in ix_descs(j + NIB - 3, sd):
                        d.start()
            return 0
        lax.fori_loop(0, NCH // NIB, chunk_body, 0)

        plsc.subcore_barrier()

        pltpu.sync_copy(
            acc.at[pl.ds(s * ZPT, ZPT)],
            out.at[pl.ds((kk * TPAD + s * ZPT), ZPT)])

    if do_hist:
        wid = c * NS + s
        pltpu.sync_copy(cnt_loc, cnt_out.at[pl.ds(wid * TPAD, TPAD)])


def _make_sc_pass(nq, do_hist=False):
    out_types = [jax.ShapeDtypeStruct((NC * nq * TPAD, COLS), SCDT)]
    scratch = [
        pltpu.VMEM_SHARED((TPAD, COLS), SCDT),
        pltpu.VMEM((NBUF, CHUNK, COLS), SCDT),
        pltpu.VMEM((NIB, CHUNK), jnp.int32),
        pltpu.VMEM((NIB, CHUNK), jnp.int32),
        pltpu.VMEM((32, COLS), SCDT),
    ]
    if do_hist:
        out_types.append(jax.ShapeDtypeStruct((NC * NS * TPAD,), jnp.float32))
        scratch.append(pltpu.VMEM((CHUNK,), jnp.int32))
        scratch.append(pltpu.VMEM((TPAD,), jnp.float32))
    scratch += [
        pltpu.SemaphoreType.DMA((NBUF,)),
        pltpu.SemaphoreType.DMA((NIB,)),
        pltpu.SemaphoreType.DMA((NBUF,)),
    ]
    return pl.kernel(
        functools.partial(_sc_pass_body, nq, do_hist),
        out_type=tuple(out_types) if do_hist else out_types[0],
        mesh=_mesh(),
        scratch_types=scratch,
        compiler_params=pltpu.CompilerParams(**_SC_PARAMS),
    )


def _full(shape):
    return pl.BlockSpec(shape, lambda i: tuple(0 for _ in shape))


def _sum_specs(nchunks):
    return [
        pl.BlockSpec((nchunks, _BN, COLS),
                     functools.partial(lambda r, i: (0, r * NB + i, 0), r))
        for r in range(R)
    ]


def _tc_xcat_body(x, xcat):
    xb = x[...]
    for k in range(NCK1):
        xcat[k] = xb[:, k * COLS:(k + 1) * COLS].astype(SCDT)


def _tc_hsc_body(x, s0, s1, s2, cr0, cr1, cr2, wroot, b,
                 hscat, r1root, cntB):
    xb = x[...]
    ss = (s0, s1, s2)
    craw = (cr0, cr1, cr2)
    ctx = jnp.zeros((_BN, DIN), jnp.float32)
    relcnt = jnp.zeros((_BN,), jnp.float32)
    cb = []
    for r in range(R):
        blk = craw[r][...]
        ca = blk[0]
        cbr = blk[NS]
        for w in range(1, NS):
            ca = ca + blk[w]
            cbr = cbr + blk[NS + w]
        cb.append(cbr)
        inv = 1.0 / jnp.maximum(ca, 1.0)
        sr = jnp.concatenate(
            [ss[r][k].astype(jnp.float32) for k in range(NCK1)], axis=1)
        ctx = ctx + sr * inv[:, None]
        relcnt = relcnt + jnp.minimum(ca, 1.0)
    cntB[...] = jnp.stack(cb, axis=0)
    h = xb - ctx * (1.0 / jnp.maximum(relcnt, 1.0))[:, None]
    for k in range(NCK1):
        hscat[k] = h[:, k * COLS:(k + 1) * COLS].astype(SCDT)
    r1root[...] = jnp.dot(h, wroot[...], preferred_element_type=jnp.float32) + b[...]


def _tc_z1_body(r1root, s0, s1, s2, cb, wr, g, be, wroot2, b2,
                z1cat, r2root):
    acc = r1root[...]
    ss = (s0, s1, s2)
    cbb = cb[...]
    for r in range(R):
        inv = 1.0 / jnp.maximum(cbb[r], 1.0)
        m = jnp.concatenate(
            [ss[r][k].astype(jnp.float32) for k in range(NCK1)], axis=1)
        acc = acc + jnp.dot(m * inv[:, None], wr[r],
                            preferred_element_type=jnp.float32)
    s = g[...] * RS
    z1 = jnp.maximum(acc * s + be[...], 0.0)
    for k in range(NCK2):
        z1cat[k] = z1[:, k * COLS:(k + 1) * COLS].astype(SCDT)
    r2root[...] = jnp.dot(z1, wroot2[...], preferred_element_type=jnp.float32) + b2[...]


def _tc_out_body(r2root, s0, s1, s2, cb, wr, g, be,
                 x, w1, b1, g1, be1, w2, b2, g2, be2,
                 ow, ob, logits):
    acc = r2root[...]
    ss = (s0, s1, s2)
    cbb = cb[...]
    for r in range(R):
        inv = 1.0 / jnp.maximum(cbb[r], 1.0)
        m = jnp.concatenate(
            [ss[r][k].astype(jnp.float32) for k in range(NCK2)], axis=1)
        acc = acc + jnp.dot(m * inv[:, None], wr[r],
                            preferred_element_type=jnp.float32)
    s2c = g[...] * RS
    z2 = jnp.maximum(acc * s2c + be[...], 0.0)

    hm = jnp.dot(x[...], w1[...], preferred_element_type=jnp.float32)
    sm1 = g1[...] * RS
    hm = jnp.maximum(hm * sm1 + (b1[...] * sm1 + be1[...]), 0.0)
    hm = jnp.dot(hm, w2[...], preferred_element_type=jnp.float32)
    sm2 = g2[...] * RS
    zm = jnp.maximum(hm * sm2 + (b2[...] * sm2 + be2[...]), 0.0)

    w_all = ow[...]
    out = jnp.dot(zm, w_all[:H], preferred_element_type=jnp.float32)
    out = out + jnp.dot(z2, w_all[H:], preferred_element_type=jnp.float32)
    logits[...] = out + ob[...]


def kernel(x, edge_index, edge_type, params):
    p = params
    row = edge_index[0]
    col = edge_index[1]
    et = edge_type

    pad_t = jnp.full((EPAD - E,), N, jnp.int32)
    pad_z = jnp.zeros((EPAD - E,), jnp.int32)
    tA = jnp.concatenate([et * NPAD + row, pad_t]).reshape(NS, NCH, CHUNK)
    tB = jnp.concatenate([et * NPAD + col, pad_t]).reshape(NS, NCH, CHUNK)
    col_p = jnp.concatenate([col, pad_z])
    row_p = jnp.concatenate([row, pad_z])
    gidxA = jnp.concatenate([col_p + k * NPAD for k in range(NCK1)])
    gidxB1 = jnp.concatenate([row_p + k * NPAD for k in range(NCK1)])
    gidxB2 = jnp.concatenate([row_p + k * NPAD for k in range(NCK2)])

    (b1, g1, be1, b2, g2, be2, rb1, rb2) = [
        a.reshape(1, H) for a in (
            p['mlp_b1'], p['mlp_g1'], p['mlp_be1'],
            p['mlp_b2'], p['mlp_g2'], p['mlp_be2'],
            p['rgcn1_b'], p['rgcn2_b'])]
    (bn1g, bn1b, bn2g, bn2b) = [
        a.reshape(1, H) for a in (p['bn1_g'], p['bn1_be'], p['bn2_g'], p['bn2_be'])]

    xcat = pl.pallas_call(
        _tc_xcat_body,
        grid=(NB,),
        in_specs=[pl.BlockSpec((_BN, DIN), lambda i: (i, 0))],
        out_specs=pl.BlockSpec((NCK1, _BN, COLS), lambda i: (0, i, 0)),
        out_shape=jax.ShapeDtypeStruct((NCK1, NPAD, COLS), SCDT),
    )(x)

    outA, cnt_raw = _make_sc_pass(NCK1 // NC, True)(
        xcat.reshape(NCK1 * NPAD, COLS), gidxA, tA, tB)
    outA = outA.reshape(NCK1, TPAD, COLS)
    cnt_raw = cnt_raw.reshape(2 * NS, TPAD)

    craw_specs = [
        pl.BlockSpec((2 * NS, _BN),
                     functools.partial(lambda r, i: (0, r * NB + i), r))
        for r in range(R)
    ]
    hscat, r1root, cntB = pl.pallas_call(
        _tc_hsc_body,
        grid=(NB,),
        in_specs=[pl.BlockSpec((_BN, DIN), lambda i: (i, 0))] +
                 _sum_specs(NCK1) + craw_specs +
                 [_full((DIN, H)), _full((1, H))],
        out_specs=[pl.BlockSpec((NCK1, _BN, COLS), lambda i: (0, i, 0)),
                   pl.BlockSpec((_BN, H), lambda i: (i, 0)),
                   pl.BlockSpec((R, _BN), lambda i: (0, i))],
        out_shape=[jax.ShapeDtypeStruct((NCK1, NPAD, COLS), SCDT),
                   jax.ShapeDtypeStruct((NPAD, H), jnp.float32),
                   jax.ShapeDtypeStruct((R, NPAD), jnp.float32)],
    )(x, outA, outA, outA, cnt_raw, cnt_raw, cnt_raw, p['rgcn1_Wroot'], rb1)

    outB1 = _make_sc_pass(NCK1 // NC)(
        hscat.reshape(NCK1 * NPAD, COLS), gidxB1, tB)
    outB1 = outB1.reshape(NCK1, TPAD, COLS)

    cntB_spec = pl.BlockSpec((R, _BN), lambda i: (0, i))

    z1cat, r2root = pl.pallas_call(
        _tc_z1_body,
        grid=(NB,),
        in_specs=[pl.BlockSpec((_BN, H), lambda i: (i, 0))] +
                 _sum_specs(NCK1) + [cntB_spec] +
                 [_full((R, DIN, H)), _full((1, H)), _full((1, H)),
                  _full((H, H)), _full((1, H))],
        out_specs=[pl.BlockSpec((NCK2, _BN, COLS), lambda i: (0, i, 0)),
                   pl.BlockSpec((_BN, H), lambda i: (i, 0))],
        out_shape=[jax.ShapeDtypeStruct((NCK2, NPAD, COLS), SCDT),
                   jax.ShapeDtypeStruct((NPAD, H), jnp.float32)],
    )(r1root, outB1, outB1, outB1, cntB,
      p['rgcn1_Wr'], bn1g, bn1b, p['rgcn2_Wroot'], rb2)

    outB2 = _make_sc_pass(NCK2 // NC)(
        z1cat.reshape(NCK2 * NPAD, COLS), gidxB2, tB)
    outB2 = outB2.reshape(NCK2, TPAD, COLS)

    logits = pl.pallas_call(
        _tc_out_body,
        grid=(NB,),
        in_specs=[pl.BlockSpec((_BN, H), lambda i: (i, 0))] +
                 _sum_specs(NCK2) + [cntB_spec] +
                 [_full((R, H, H)), _full((1, H)), _full((1, H)),
                  pl.BlockSpec((_BN, DIN), lambda i: (i, 0)),
                  _full((DIN, H))] + [_full((1, H))] * 3 +
                 [_full((H, H))] + [_full((1, H))] * 3 +
                 [_full((2 * H, 1)), _full((1, 1))],
        out_specs=pl.BlockSpec((_BN, 1), lambda i: (i, 0)),
        out_shape=jax.ShapeDtypeStruct((N, 1), jnp.float32),
    )(r2root, outB2, outB2, outB2, cntB,
      p['rgcn2_Wr'], bn2g, bn2b,
      x, p['mlp_W1'], b1, g1, be1, p['mlp_W2'], b2, g2, be2,
      p['out_W'], p['out_b'].reshape(1, 1))

    return logits[:, 0]
